# baseline (speedup 1.0000x reference)
import jax
import jax.numpy as jnp
from jax import lax
from jax.experimental import pallas as pl
from jax.experimental.pallas import tpu as pltpu
from jax.experimental.pallas import tpu_sc as plsc

BATCH = 16384
EMB = 32
NC = 2
NS = 16
C_V = 100000
C_TF = C_V // 128
C_TAIL = C_V - C_TF * 128
D_V = 1000
D_TF = D_V // 128
D_TAIL = D_V - D_TF * 128
VQ = 64
FPQ = 22
HB = BATCH // 4
GCH = 1024
UNROLL = GCH // 16


def _sc_body(d_hi, d_lo, c0_hi, c0_lo, c1_hi, c1_lo,
             dT, c0T, c1T, tails, out,
             shared, row_v, tail_v, hi_v, lo_v, out_v, sem):
    c = lax.axis_index("c")
    s = lax.axis_index("s")
    e = 16 * c + s
    bl = s // 8
    s8 = s % 8
    def stage(st, tab, tf, tail, toff, hi, lo):
        for bp in range(2):
            for qp in range((tf + VQ - 1) // VQ):
                lo_t = qp * VQ
                npass = min(VQ, tf - lo_t)
                if npass <= 0:
                    continue

                def fetch(t, carry, bp=bp, lo_t=lo_t, npass=npass):
                    tg = s * FPQ + t

                    @pl.when(tg < npass)
                    def _():
                        pltpu.sync_copy(
                            tab.at[pl.ds(8 * (2 * c + bp), 8),
                                   pl.ds(128 * lo_t + 128 * tg, 128)],
                            shared.at[:, tg, :])
                    return carry

                lax.fori_loop(0, min(FPQ, npass), fetch, 0)
                plsc.subcore_barrier()

                @pl.when(bl == bp)
                def _(lo_t=lo_t, npass=npass):
                    pltpu.sync_copy(shared.at[s8, pl.ds(0, npass), :],
                                    row_v.at[pl.ds(lo_t, npass), :])
                plsc.subcore_barrier()
        pltpu.sync_copy(tails.at[pl.ds(toff + e * tail, tail)],
                        tail_v.at[pl.ds(0, tail)])
        for h in range(4):
            pltpu.sync_copy(hi.at[pl.ds(h * HB, HB)], hi_v)
            pltpu.sync_copy(lo.at[pl.ds(h * HB, HB)], lo_v)

            def chunk(t, carry):
                for j in range(UNROLL):
                    o = t * GCH + j * 16
                    iv_hi = hi_v[pl.ds(o, 16)]
                    iv_lo = lo_v[pl.ds(o, 16)]
                    g = plsc.load_gather(row_v, [iv_hi, iv_lo])
                    gt = plsc.load_gather(tail_v, [iv_lo])
                    g = jnp.where(iv_hi == tf, gt, g)
                    out_v[t * 8 + j // 8, pl.ds((j % 8) * 16, 16)] = g
                return carry

            lax.fori_loop(0, HB // GCH, chunk, 0)
            pltpu.sync_copy(out_v,
                            out.at[st, pl.ds(e * 128 + h * 32, 32), :])
        plsc.subcore_barrier()

    stage(0, dT, D_TF, D_TAIL, 0, d_hi, d_lo)
    stage(1, c0T, C_TF, C_TAIL, D_TAIL * 32, c0_hi, c0_lo)
    stage(2, c1T, C_TF, C_TAIL, D_TAIL * 32 + C_TAIL * 32, c1_hi, c1_lo)


def _make_sc_gather():
    return pl.kernel(
        _sc_body,
        mesh=plsc.VectorSubcoreMesh(core_axis_name="c", subcore_axis_name="s"),
        compiler_params=pltpu.CompilerParams(
            use_tc_tiling_on_sc=True, needs_layout_passes=False),
        out_type=jax.ShapeDtypeStruct((3, BATCH * EMB // 128, 128),
                                      jnp.float32),
        scratch_types=[
            pltpu.VMEM_SHARED((8, VQ, 128), jnp.float32),
            pltpu.VMEM((C_TF + 1, 128), jnp.float32),
            pltpu.VMEM((128,), jnp.float32),
            pltpu.VMEM((HB,), jnp.int32),
            pltpu.VMEM((HB,), jnp.int32),
            pltpu.VMEM((32, 128), jnp.float32),
            pltpu.SemaphoreType.DMA,
        ],
    )


BB = 2048


def _tc_body(didx, c0idx, c1idx, dT_ref, c0T_ref, c1T_ref,
             w0t, b0, w1t, b1, w2t, b2, o0T, o1, o2T):
    md = (didx[...] != 0).astype(jnp.float32)
    m0 = (c0idx[...] != 0).astype(jnp.float32)
    m1 = (c1idx[...] != 0).astype(jnp.float32)
    ch = c0T_ref[...] * m0 + c1T_ref[...] * m1
    eT = (dT_ref[...] * md) * (ch * 0.5)
    dn = (((0,), (0,)), ((), ()))
    o0T[...] = lax.dot_general(w0t[...], eT, dn,
                               preferred_element_type=jnp.float32) + b0[...]
    o1[...] = lax.dot_general(eT, w1t[...], dn,
                              preferred_element_type=jnp.float32) + b1[...]
    o2T[...] = lax.dot_general(w2t[...], eT, dn,
                               preferred_element_type=jnp.float32) + b2[...]


def _tc_call(didx, c0idx, c1idx, dT_g, c0T_g, c1T_g,
             W0, b0, W1, b1, W2, b2):
    t0, t1, t2 = W0.shape[0], W1.shape[0], W2.shape[0]
    f32 = jnp.float32
    embT_spec = pl.BlockSpec((EMB, BB), lambda i: (0, i))
    idx_spec = pl.BlockSpec((1, BB), lambda i: (0, i))
    full = lambda shape: pl.BlockSpec(shape, lambda i: (0, 0))
    return pl.pallas_call(
        _tc_body,
        grid=(BATCH // BB,),
        in_specs=[idx_spec, idx_spec, idx_spec,
                  embT_spec, embT_spec, embT_spec,
                  full((EMB, t0)), full((t0, 1)),
                  full((EMB, t1)), full((1, t1)),
                  full((EMB, t2)), full((t2, 1))],
        out_specs=[pl.BlockSpec((t0, BB), lambda i: (0, i)),
                   pl.BlockSpec((BB, t1), lambda i: (i, 0)),
                   pl.BlockSpec((t2, BB), lambda i: (0, i))],
        out_shape=[jax.ShapeDtypeStruct((t0, BATCH), f32),
                   jax.ShapeDtypeStruct((BATCH, t1), f32),
                   jax.ShapeDtypeStruct((t2, BATCH), f32)],
    )(didx, c0idx, c1idx, dT_g, c0T_g, c1T_g,
      W0.T, b0.reshape(t0, 1), W1.T, b1.reshape(1, t1),
      W2.T, b2.reshape(t2, 1))


def kernel(dialects, chars, d_emb0, c_emb0, c_emb1, W0, b0, W1, b1, W2, b2):
    d_idx = dialects[:, 0].astype(jnp.int32)
    c0_idx = chars[:, 0].astype(jnp.int32)
    c1_idx = chars[:, 1].astype(jnp.int32)
    tails = jnp.concatenate([
        d_emb0[D_TF * 128:, :].T.reshape(-1),
        c_emb0[C_TF * 128:, :].T.reshape(-1),
        c_emb1[C_TF * 128:, :].T.reshape(-1)])
    g = _make_sc_gather()(
        d_idx >> 7, d_idx & 127, c0_idx >> 7, c0_idx & 127,
        c1_idx >> 7, c1_idx & 127, d_emb0.T, c_emb0.T, c_emb1.T, tails)
    d_g, c0_g, c1_g = g[0], g[1], g[2]
    o0T, o1, o2T = _tc_call(
        d_idx.reshape(1, BATCH), c0_idx.reshape(1, BATCH),
        c1_idx.reshape(1, BATCH),
        d_g.reshape(EMB, BATCH), c0_g.reshape(EMB, BATCH),
        c1_g.reshape(EMB, BATCH),
        W0, b0, W1, b1, W2, b2)
    return (o0T.T, o1, o2T.T)


# speedup vs baseline: 6.5254x; 6.5254x over previous
import jax
import jax.numpy as jnp
from jax import lax
from jax.experimental import pallas as pl
from jax.experimental.pallas import tpu as pltpu
from jax.experimental.pallas import tpu_sc as plsc

BATCH = 16384
EMB = 32
NC = 2
NS = 16
C_V = 100000
C_TF = C_V // 128
C_TAIL = C_V - C_TF * 128
D_V = 1000
D_TF = D_V // 128
D_TAIL = D_V - D_TF * 128
VQ = 272
HB = BATCH // 4
GCH = 1024
UNROLL = GCH // 16


def _sc_body(d_hi, d_lo, c0_hi, c0_lo, c1_hi, c1_lo,
             dT, c0T, c1T, tails, out,
             shared, row_v, tail_v, hi_v, lo_v, out_v, sem):
    c = lax.axis_index("c")
    s = lax.axis_index("s")
    e = 16 * c + s
    bl = s // 8
    s8 = s % 8
    def stage(st, tab, tf, tail, toff, hi, lo):
        for bp in range(2):
            for qp in range((tf + VQ - 1) // VQ):
                lo_t = qp * VQ
                npass = min(VQ, tf - lo_t)
                nfetch = (npass + 15) // 16
                copies = []
                for t in range(nfetch):
                    tg = jnp.minimum(s * nfetch + t, npass - 1)
                    copies.append(pltpu.async_copy(
                        tab.at[pl.ds(8 * (2 * c + bp), 8),
                               pl.ds(128 * lo_t + 128 * tg, 128)],
                        shared.at[:, tg, :], sem))
                for cp in copies:
                    cp.wait()
                plsc.subcore_barrier()

                @pl.when(bl == bp)
                def _(lo_t=lo_t, npass=npass):
                    pltpu.sync_copy(shared.at[s8, pl.ds(0, npass), :],
                                    row_v.at[pl.ds(lo_t, npass), :])
                plsc.subcore_barrier()
        pltpu.sync_copy(tails.at[pl.ds(toff + e * tail, tail)],
                        tail_v.at[pl.ds(0, tail)])
        def section(h, carry):
            pltpu.sync_copy(hi.at[pl.ds(h * HB, HB)], hi_v)
            pltpu.sync_copy(lo.at[pl.ds(h * HB, HB)], lo_v)

            def chunk(t, carry2):
                for j in range(UNROLL):
                    o = t * GCH + j * 16
                    iv_hi = hi_v[pl.ds(o, 16)]
                    iv_lo = lo_v[pl.ds(o, 16)]
                    g = plsc.load_gather(row_v, [iv_hi, iv_lo])
                    gt = plsc.load_gather(tail_v, [iv_lo])
                    g = jnp.where(iv_hi == tf, gt, g)
                    out_v[t * 8 + j // 8, pl.ds((j % 8) * 16, 16)] = g
                return carry2

            lax.fori_loop(0, HB // GCH, chunk, 0)
            pltpu.sync_copy(out_v,
                            out.at[st, pl.ds(e * 128 + h * 32, 32), :])
            return carry

        lax.fori_loop(0, BATCH // HB, section, 0)
        plsc.subcore_barrier()

    stage(0, dT, D_TF, D_TAIL, 0, d_hi, d_lo)
    stage(1, c0T, C_TF, C_TAIL, D_TAIL * 32, c0_hi, c0_lo)
    stage(2, c1T, C_TF, C_TAIL, D_TAIL * 32 + C_TAIL * 32, c1_hi, c1_lo)


def _make_sc_gather():
    return pl.kernel(
        _sc_body,
        mesh=plsc.VectorSubcoreMesh(core_axis_name="c", subcore_axis_name="s"),
        compiler_params=pltpu.CompilerParams(
            use_tc_tiling_on_sc=True, needs_layout_passes=False),
        out_type=jax.ShapeDtypeStruct((3, BATCH * EMB // 128, 128),
                                      jnp.float32),
        scratch_types=[
            pltpu.VMEM_SHARED((8, VQ, 128), jnp.float32),
            pltpu.VMEM((C_TF + 1, 128), jnp.float32),
            pltpu.VMEM((128,), jnp.float32),
            pltpu.VMEM((HB,), jnp.int32),
            pltpu.VMEM((HB,), jnp.int32),
            pltpu.VMEM((32, 128), jnp.float32),
            pltpu.SemaphoreType.DMA,
        ],
    )


BB = 2048


def _tc_body(didx, c0idx, c1idx, dT_ref, c0T_ref, c1T_ref,
             w0t, b0, w1t, b1, w2t, b2, o0T, o1, o2T):
    md = (didx[...] != 0).astype(jnp.float32)
    m0 = (c0idx[...] != 0).astype(jnp.float32)
    m1 = (c1idx[...] != 0).astype(jnp.float32)
    ch = c0T_ref[...] * m0 + c1T_ref[...] * m1
    eT = (dT_ref[...] * md) * (ch * 0.5)
    dn = (((0,), (0,)), ((), ()))
    o0T[...] = lax.dot_general(w0t[...], eT, dn,
                               preferred_element_type=jnp.float32) + b0[...]
    o1[...] = lax.dot_general(eT, w1t[...], dn,
                              preferred_element_type=jnp.float32) + b1[...]
    o2T[...] = lax.dot_general(w2t[...], eT, dn,
                               preferred_element_type=jnp.float32) + b2[...]


def _tc_call(didx, c0idx, c1idx, dT_g, c0T_g, c1T_g,
             W0, b0, W1, b1, W2, b2):
    t0, t1, t2 = W0.shape[0], W1.shape[0], W2.shape[0]
    f32 = jnp.float32
    embT_spec = pl.BlockSpec((EMB, BB), lambda i: (0, i))
    idx_spec = pl.BlockSpec((1, BB), lambda i: (0, i))
    full = lambda shape: pl.BlockSpec(shape, lambda i: (0, 0))
    return pl.pallas_call(
        _tc_body,
        grid=(BATCH // BB,),
        in_specs=[idx_spec, idx_spec, idx_spec,
                  embT_spec, embT_spec, embT_spec,
                  full((EMB, t0)), full((t0, 1)),
                  full((EMB, t1)), full((1, t1)),
                  full((EMB, t2)), full((t2, 1))],
        out_specs=[pl.BlockSpec((t0, BB), lambda i: (0, i)),
                   pl.BlockSpec((BB, t1), lambda i: (i, 0)),
                   pl.BlockSpec((t2, BB), lambda i: (0, i))],
        out_shape=[jax.ShapeDtypeStruct((t0, BATCH), f32),
                   jax.ShapeDtypeStruct((BATCH, t1), f32),
                   jax.ShapeDtypeStruct((t2, BATCH), f32)],
    )(didx, c0idx, c1idx, dT_g, c0T_g, c1T_g,
      W0.T, b0.reshape(t0, 1), W1.T, b1.reshape(1, t1),
      W2.T, b2.reshape(t2, 1))


def kernel(dialects, chars, d_emb0, c_emb0, c_emb1, W0, b0, W1, b1, W2, b2):
    d_idx = dialects[:, 0].astype(jnp.int32)
    c0_idx = chars[:, 0].astype(jnp.int32)
    c1_idx = chars[:, 1].astype(jnp.int32)
    tails = jnp.concatenate([
        d_emb0[D_TF * 128:, :].T.reshape(-1),
        c_emb0[C_TF * 128:, :].T.reshape(-1),
        c_emb1[C_TF * 128:, :].T.reshape(-1)])
    g = _make_sc_gather()(
        d_idx >> 7, d_idx & 127, c0_idx >> 7, c0_idx & 127,
        c1_idx >> 7, c1_idx & 127, d_emb0.T, c_emb0.T, c_emb1.T, tails)
    d_g, c0_g, c1_g = g[0], g[1], g[2]
    o0T, o1, o2T = _tc_call(
        d_idx.reshape(1, BATCH), c0_idx.reshape(1, BATCH),
        c1_idx.reshape(1, BATCH),
        d_g.reshape(EMB, BATCH), c0_g.reshape(EMB, BATCH),
        c1_g.reshape(EMB, BATCH),
        W0, b0, W1, b1, W2, b2)
    return (o0T.T, o1, o2T.T)


# speedup vs baseline: 6.6920x; 1.0255x over previous
import jax
import jax.numpy as jnp
from jax import lax
from jax.experimental import pallas as pl
from jax.experimental.pallas import tpu as pltpu
from jax.experimental.pallas import tpu_sc as plsc

BATCH = 16384
EMB = 32
NC = 2
NS = 16
C_V = 100000
C_TF = C_V // 128
C_TAIL = C_V - C_TF * 128
D_V = 1000
D_TF = D_V // 128
D_TAIL = D_V - D_TF * 128
VQ = 272
HB = BATCH // 4
GCH = 1024
UNROLL = GCH // 16


def _sc_body(d_hi, d_lo, c0_hi, c0_lo, c1_hi, c1_lo,
             dT, c0T, c1T, tails, out,
             shared, row_v, tail_v, hi_v, lo_v, out_v, sem):
    c = lax.axis_index("c")
    s = lax.axis_index("s")
    e = 16 * c + s
    bl = s // 8
    s8 = s % 8
    def stage(st, tab, tf, tail, toff, hi, lo):
        for bp in range(2):
            for qp in range((tf + VQ - 1) // VQ):
                lo_t = qp * VQ
                npass = min(VQ, tf - lo_t)
                nfetch = (npass + 15) // 16
                copies = []
                for t in range(nfetch):
                    tg = jnp.minimum(s * nfetch + t, npass - 1)
                    copies.append(pltpu.async_copy(
                        tab.at[pl.ds(8 * (2 * c + bp), 8),
                               pl.ds(128 * lo_t + 128 * tg, 128)],
                        shared.at[:, tg, :], sem))
                for cp in copies:
                    cp.wait()
                plsc.subcore_barrier()

                @pl.when(bl == bp)
                def _(lo_t=lo_t, npass=npass):
                    pltpu.sync_copy(shared.at[s8, pl.ds(0, npass), :],
                                    row_v.at[pl.ds(lo_t, npass), :])
                plsc.subcore_barrier()
        pltpu.sync_copy(tails.at[pl.ds(toff + e * tail, tail)],
                        tail_v.at[pl.ds(0, tail)])
        for k in range(8):
            row_v[tf, pl.ds(16 * k, 16)] = tail_v[pl.ds(16 * k, 16)]
        def section(h, carry):
            pltpu.sync_copy(hi.at[pl.ds(h * HB, HB)], hi_v)
            pltpu.sync_copy(lo.at[pl.ds(h * HB, HB)], lo_v)

            def chunk(t, carry2):
                for j in range(UNROLL):
                    o = t * GCH + j * 16
                    iv_hi = hi_v[pl.ds(o, 16)]
                    iv_lo = lo_v[pl.ds(o, 16)]
                    g = plsc.load_gather(row_v, [iv_hi, iv_lo])
                    out_v[t * 8 + j // 8, pl.ds((j % 8) * 16, 16)] = g
                return carry2

            lax.fori_loop(0, HB // GCH, chunk, 0)
            pltpu.sync_copy(out_v,
                            out.at[st, pl.ds(e * 128 + h * 32, 32), :])
            return carry

        lax.fori_loop(0, BATCH // HB, section, 0)
        plsc.subcore_barrier()

    stage(0, dT, D_TF, D_TAIL, 0, d_hi, d_lo)
    stage(1, c0T, C_TF, C_TAIL, D_TAIL * 32, c0_hi, c0_lo)
    stage(2, c1T, C_TF, C_TAIL, D_TAIL * 32 + C_TAIL * 32, c1_hi, c1_lo)


def _make_sc_gather():
    return pl.kernel(
        _sc_body,
        mesh=plsc.VectorSubcoreMesh(core_axis_name="c", subcore_axis_name="s"),
        compiler_params=pltpu.CompilerParams(
            use_tc_tiling_on_sc=True, needs_layout_passes=False),
        out_type=jax.ShapeDtypeStruct((3, BATCH * EMB // 128, 128),
                                      jnp.float32),
        scratch_types=[
            pltpu.VMEM_SHARED((8, VQ, 128), jnp.float32),
            pltpu.VMEM((C_TF + 1, 128), jnp.float32),
            pltpu.VMEM((128,), jnp.float32),
            pltpu.VMEM((HB,), jnp.int32),
            pltpu.VMEM((HB,), jnp.int32),
            pltpu.VMEM((32, 128), jnp.float32),
            pltpu.SemaphoreType.DMA,
        ],
    )


BB = 2048


def _tc_body(didx, c0idx, c1idx, dT_ref, c0T_ref, c1T_ref,
             w0t, b0, w1t, b1, w2t, b2, o0T, o1, o2T):
    md = (didx[...] != 0).astype(jnp.float32)
    m0 = (c0idx[...] != 0).astype(jnp.float32)
    m1 = (c1idx[...] != 0).astype(jnp.float32)
    ch = c0T_ref[...] * m0 + c1T_ref[...] * m1
    eT = (dT_ref[...] * md) * (ch * 0.5)
    dn = (((0,), (0,)), ((), ()))
    o0T[...] = lax.dot_general(w0t[...], eT, dn,
                               preferred_element_type=jnp.float32) + b0[...]
    o1[...] = lax.dot_general(eT, w1t[...], dn,
                              preferred_element_type=jnp.float32) + b1[...]
    o2T[...] = lax.dot_general(w2t[...], eT, dn,
                               preferred_element_type=jnp.float32) + b2[...]


def _tc_call(didx, c0idx, c1idx, dT_g, c0T_g, c1T_g,
             W0, b0, W1, b1, W2, b2):
    t0, t1, t2 = W0.shape[0], W1.shape[0], W2.shape[0]
    f32 = jnp.float32
    embT_spec = pl.BlockSpec((EMB, BB), lambda i: (0, i))
    idx_spec = pl.BlockSpec((1, BB), lambda i: (0, i))
    full = lambda shape: pl.BlockSpec(shape, lambda i: (0, 0))
    return pl.pallas_call(
        _tc_body,
        grid=(BATCH // BB,),
        in_specs=[idx_spec, idx_spec, idx_spec,
                  embT_spec, embT_spec, embT_spec,
                  full((EMB, t0)), full((t0, 1)),
                  full((EMB, t1)), full((1, t1)),
                  full((EMB, t2)), full((t2, 1))],
        out_specs=[pl.BlockSpec((t0, BB), lambda i: (0, i)),
                   pl.BlockSpec((BB, t1), lambda i: (i, 0)),
                   pl.BlockSpec((t2, BB), lambda i: (0, i))],
        out_shape=[jax.ShapeDtypeStruct((t0, BATCH), f32),
                   jax.ShapeDtypeStruct((BATCH, t1), f32),
                   jax.ShapeDtypeStruct((t2, BATCH), f32)],
    )(didx, c0idx, c1idx, dT_g, c0T_g, c1T_g,
      W0.T, b0.reshape(t0, 1), W1.T, b1.reshape(1, t1),
      W2.T, b2.reshape(t2, 1))


def kernel(dialects, chars, d_emb0, c_emb0, c_emb1, W0, b0, W1, b1, W2, b2):
    d_idx = dialects[:, 0].astype(jnp.int32)
    c0_idx = chars[:, 0].astype(jnp.int32)
    c1_idx = chars[:, 1].astype(jnp.int32)
    tails = jnp.concatenate([
        d_emb0[D_TF * 128:, :].T.reshape(-1),
        c_emb0[C_TF * 128:, :].T.reshape(-1),
        c_emb1[C_TF * 128:, :].T.reshape(-1)])
    g = _make_sc_gather()(
        d_idx >> 7, d_idx & 127, c0_idx >> 7, c0_idx & 127,
        c1_idx >> 7, c1_idx & 127, d_emb0.T, c_emb0.T, c_emb1.T, tails)
    d_g, c0_g, c1_g = g[0], g[1], g[2]
    o0T, o1, o2T = _tc_call(
        d_idx.reshape(1, BATCH), c0_idx.reshape(1, BATCH),
        c1_idx.reshape(1, BATCH),
        d_g.reshape(EMB, BATCH), c0_g.reshape(EMB, BATCH),
        c1_g.reshape(EMB, BATCH),
        W0, b0, W1, b1, W2, b2)
    return (o0T.T, o1, o2T.T)


# speedup vs baseline: 6.9233x; 1.0346x over previous
import jax
import jax.numpy as jnp
from jax import lax
from jax.experimental import pallas as pl
from jax.experimental.pallas import tpu as pltpu
from jax.experimental.pallas import tpu_sc as plsc

BATCH = 16384
EMB = 32
NC = 2
NS = 16
C_V = 100000
C_TF = C_V // 128
C_TAIL = C_V - C_TF * 128
D_V = 1000
D_TF = D_V // 128
D_TAIL = D_V - D_TF * 128
VQ = 272
HB = BATCH // 4
GCH = 1024
UNROLL = GCH // 16


def _sc_body(d_hi, d_lo, c0_hi, c0_lo, c1_hi, c1_lo,
             dT, c0T, c1T, tails, out,
             shared, row_v, tail_v, hi_v, lo_v, out_v, sem):
    c = lax.axis_index("c")
    s = lax.axis_index("s")
    e = 16 * c + s
    bl = s // 8
    s8 = s % 8
    def stage(st, tab, tf, tail, toff, hi, lo):
        for bp in range(2):
            for qp in range((tf + VQ - 1) // VQ):
                lo_t = qp * VQ
                npass = min(VQ, tf - lo_t)
                nfetch = (npass + 15) // 16
                copies = []
                for t in range(nfetch):
                    tg = jnp.minimum(s * nfetch + t, npass - 1)
                    copies.append(pltpu.async_copy(
                        tab.at[pl.ds(8 * (2 * c + bp), 8),
                               pl.ds(128 * lo_t + 128 * tg, 128)],
                        shared.at[:, tg, :], sem))
                for cp in copies:
                    cp.wait()
                plsc.subcore_barrier()

                @pl.when(bl == bp)
                def _(lo_t=lo_t, npass=npass):
                    pltpu.sync_copy(shared.at[s8, pl.ds(0, npass), :],
                                    row_v.at[pl.ds(lo_t, npass), :])
                plsc.subcore_barrier()
        pltpu.sync_copy(tails.at[pl.ds(toff + e * tail, tail)],
                        tail_v.at[pl.ds(0, tail)])
        for k in range(8):
            row_v[tf, pl.ds(16 * k, 16)] = tail_v[pl.ds(16 * k, 16)]
        def section(h, carry):
            pltpu.sync_copy(hi.at[pl.ds(h * HB, HB)], hi_v)
            pltpu.sync_copy(lo.at[pl.ds(h * HB, HB)], lo_v)

            def chunk(t, carry2):
                for j in range(UNROLL):
                    o = t * GCH + j * 16
                    iv_hi = hi_v[pl.ds(o, 16)]
                    iv_lo = lo_v[pl.ds(o, 16)]
                    g = plsc.load_gather(row_v, [iv_hi, iv_lo])
                    out_v[t * 8 + j // 8, pl.ds((j % 8) * 16, 16)] = g
                return carry2

            lax.fori_loop(0, HB // GCH, chunk, 0)
            pltpu.sync_copy(
                out_v, out.at[st, e // 8, pl.ds(h * 32, 32), e % 8, :])
            return carry

        lax.fori_loop(0, BATCH // HB, section, 0)
        plsc.subcore_barrier()

    stage(0, dT, D_TF, D_TAIL, 0, d_hi, d_lo)
    stage(1, c0T, C_TF, C_TAIL, D_TAIL * 32, c0_hi, c0_lo)
    stage(2, c1T, C_TF, C_TAIL, D_TAIL * 32 + C_TAIL * 32, c1_hi, c1_lo)


def _make_sc_gather():
    return pl.kernel(
        _sc_body,
        mesh=plsc.VectorSubcoreMesh(core_axis_name="c", subcore_axis_name="s"),
        compiler_params=pltpu.CompilerParams(
            use_tc_tiling_on_sc=True, needs_layout_passes=False),
        out_type=jax.ShapeDtypeStruct((3, 4, BATCH // 128, 8, 128),
                                      jnp.float32),
        scratch_types=[
            pltpu.VMEM_SHARED((8, VQ, 128), jnp.float32),
            pltpu.VMEM((C_TF + 1, 128), jnp.float32),
            pltpu.VMEM((128,), jnp.float32),
            pltpu.VMEM((HB,), jnp.int32),
            pltpu.VMEM((HB,), jnp.int32),
            pltpu.VMEM((32, 128), jnp.float32),
            pltpu.SemaphoreType.DMA,
        ],
    )


BB = 2048


def _tc_body(didx, c0idx, c1idx, dT_ref, c0T_ref, c1T_ref,
             w0t, b0, w1t, b1, w2t, b2, o0T, o1, o2T):
    md = (didx[...] != 0).astype(jnp.float32)
    m0 = (c0idx[...] != 0).astype(jnp.float32)
    m1 = (c1idx[...] != 0).astype(jnp.float32)
    ch = c0T_ref[...] * m0 + c1T_ref[...] * m1
    eT = (dT_ref[...] * md) * (ch * 0.5)
    dn = (((0,), (0,)), ((), ()))
    o0T[...] = lax.dot_general(w0t[...], eT, dn,
                               preferred_element_type=jnp.float32) + b0[...]
    o1[...] = lax.dot_general(eT, w1t[...], dn,
                              preferred_element_type=jnp.float32) + b1[...]
    o2T[...] = lax.dot_general(w2t[...], eT, dn,
                               preferred_element_type=jnp.float32) + b2[...]


def _tc_call(didx, c0idx, c1idx, dT_g, c0T_g, c1T_g,
             W0, b0, W1, b1, W2, b2):
    t0, t1, t2 = W0.shape[0], W1.shape[0], W2.shape[0]
    f32 = jnp.float32
    embT_spec = pl.BlockSpec((EMB, BB), lambda i: (0, i))
    idx_spec = pl.BlockSpec((1, BB), lambda i: (0, i))
    full = lambda shape: pl.BlockSpec(shape, lambda i: (0, 0))
    return pl.pallas_call(
        _tc_body,
        grid=(BATCH // BB,),
        in_specs=[idx_spec, idx_spec, idx_spec,
                  embT_spec, embT_spec, embT_spec,
                  full((EMB, t0)), full((t0, 1)),
                  full((EMB, t1)), full((1, t1)),
                  full((EMB, t2)), full((t2, 1))],
        out_specs=[pl.BlockSpec((t0, BB), lambda i: (0, i)),
                   pl.BlockSpec((BB, t1), lambda i: (i, 0)),
                   pl.BlockSpec((t2, BB), lambda i: (0, i))],
        out_shape=[jax.ShapeDtypeStruct((t0, BATCH), f32),
                   jax.ShapeDtypeStruct((BATCH, t1), f32),
                   jax.ShapeDtypeStruct((t2, BATCH), f32)],
    )(didx, c0idx, c1idx, dT_g, c0T_g, c1T_g,
      W0.T, b0.reshape(t0, 1), W1.T, b1.reshape(1, t1),
      W2.T, b2.reshape(t2, 1))


def kernel(dialects, chars, d_emb0, c_emb0, c_emb1, W0, b0, W1, b1, W2, b2):
    d_idx = dialects[:, 0].astype(jnp.int32)
    c0_idx = chars[:, 0].astype(jnp.int32)
    c1_idx = chars[:, 1].astype(jnp.int32)
    tails = jnp.concatenate([
        d_emb0[D_TF * 128:, :].T.reshape(-1),
        c_emb0[C_TF * 128:, :].T.reshape(-1),
        c_emb1[C_TF * 128:, :].T.reshape(-1)])
    g = _make_sc_gather()(
        d_idx >> 7, d_idx & 127, c0_idx >> 7, c0_idx & 127,
        c1_idx >> 7, c1_idx & 127, d_emb0.T, c_emb0.T, c_emb1.T, tails)
    d_g, c0_g, c1_g = (
        g[t].transpose(0, 2, 1, 3).reshape(EMB, BATCH) for t in range(3))
    o0T, o1, o2T = _tc_call(
        d_idx.reshape(1, BATCH), c0_idx.reshape(1, BATCH),
        c1_idx.reshape(1, BATCH), d_g, c0_g, c1_g,
        W0, b0, W1, b1, W2, b2)
    return (o0T.T, o1, o2T.T)


# speedup vs baseline: 7.4986x; 1.0831x over previous
import jax
import jax.numpy as jnp
from jax import lax
from jax.experimental import pallas as pl
from jax.experimental.pallas import tpu as pltpu
from jax.experimental.pallas import tpu_sc as plsc

BATCH = 16384
EMB = 32
NC = 2
NS = 16
C_V = 100000
C_TF = C_V // 128
C_TAIL = C_V - C_TF * 128
D_V = 1000
D_TF = D_V // 128
D_TAIL = D_V - D_TF * 128
VQ = 272
HB = BATCH // 4
GCH = 1024
UNROLL = GCH // 16


def _sc_body(d_hi, d_lo, c0_hi, c0_lo, c1_hi, c1_lo,
             dT, c0T, c1T, tails, out,
             shared, row_v, tail_v, hi_v, lo_v, out_v, sem):
    c = lax.axis_index("c")
    s = lax.axis_index("s")
    e = 16 * c + s
    bl = s // 8
    s8 = s % 8
    def stage(st, tab, tf, tail, toff, hi, lo):
        for bp in range(2):
            for qp in range((tf + VQ - 1) // VQ):
                lo_t = qp * VQ
                npass = min(VQ, tf - lo_t)
                nfetch = (npass + 15) // 16
                copies = []
                for t in range(nfetch):
                    tg = jnp.minimum(s * nfetch + t, npass - 1)
                    copies.append(pltpu.async_copy(
                        tab.at[pl.ds(8 * (2 * c + bp), 8),
                               pl.ds(128 * lo_t + 128 * tg, 128)],
                        shared.at[:, tg, :], sem))
                for cp in copies:
                    cp.wait()
                plsc.subcore_barrier()

                @pl.when(bl == bp)
                def _(lo_t=lo_t, npass=npass):
                    pltpu.sync_copy(shared.at[s8, pl.ds(0, npass), :],
                                    row_v.at[pl.ds(lo_t, npass), :])
                plsc.subcore_barrier()
        pltpu.sync_copy(tails.at[pl.ds(toff + e * tail, tail)],
                        tail_v.at[pl.ds(0, tail)])
        for k in range(8):
            row_v[tf, pl.ds(16 * k, 16)] = tail_v[pl.ds(16 * k, 16)]
        def section(h, carry):
            pltpu.sync_copy(hi.at[pl.ds(h * HB, HB)], hi_v)
            pltpu.sync_copy(lo.at[pl.ds(h * HB, HB)], lo_v)

            @plsc.parallel_loop(0, HB // 16, 1, unroll=8)
            def gather_one(i):
                o = i * 16
                iv_hi = hi_v[pl.ds(o, 16)]
                iv_lo = lo_v[pl.ds(o, 16)]
                g = plsc.load_gather(row_v, [iv_hi, iv_lo])
                out_v[o // 128, pl.ds(o % 128, 16)] = g

            pltpu.sync_copy(
                out_v, out.at[st, e // 8, pl.ds(h * 32, 32), e % 8, :])
            return carry

        lax.fori_loop(0, BATCH // HB, section, 0)
        plsc.subcore_barrier()

    stage(0, dT, D_TF, D_TAIL, 0, d_hi, d_lo)
    stage(1, c0T, C_TF, C_TAIL, D_TAIL * 32, c0_hi, c0_lo)
    stage(2, c1T, C_TF, C_TAIL, D_TAIL * 32 + C_TAIL * 32, c1_hi, c1_lo)


def _make_sc_gather():
    return pl.kernel(
        _sc_body,
        mesh=plsc.VectorSubcoreMesh(core_axis_name="c", subcore_axis_name="s"),
        compiler_params=pltpu.CompilerParams(
            use_tc_tiling_on_sc=True, needs_layout_passes=False),
        out_type=jax.ShapeDtypeStruct((3, 4, BATCH // 128, 8, 128),
                                      jnp.float32),
        scratch_types=[
            pltpu.VMEM_SHARED((8, VQ, 128), jnp.float32),
            pltpu.VMEM((C_TF + 1, 128), jnp.float32),
            pltpu.VMEM((128,), jnp.float32),
            pltpu.VMEM((HB,), jnp.int32),
            pltpu.VMEM((HB,), jnp.int32),
            pltpu.VMEM((32, 128), jnp.float32),
            pltpu.SemaphoreType.DMA,
        ],
    )


BB = 2048


def _tc_body(didx, c0idx, c1idx, dT_ref, c0T_ref, c1T_ref,
             w0t, b0, w1t, b1, w2t, b2, o0T, o1, o2T):
    md = (didx[...] != 0).astype(jnp.float32)
    m0 = (c0idx[...] != 0).astype(jnp.float32)
    m1 = (c1idx[...] != 0).astype(jnp.float32)
    ch = c0T_ref[...] * m0 + c1T_ref[...] * m1
    eT = (dT_ref[...] * md) * (ch * 0.5)
    dn = (((0,), (0,)), ((), ()))
    o0T[...] = lax.dot_general(w0t[...], eT, dn,
                               preferred_element_type=jnp.float32) + b0[...]
    o1[...] = lax.dot_general(eT, w1t[...], dn,
                              preferred_element_type=jnp.float32) + b1[...]
    o2T[...] = lax.dot_general(w2t[...], eT, dn,
                               preferred_element_type=jnp.float32) + b2[...]


def _tc_call(didx, c0idx, c1idx, dT_g, c0T_g, c1T_g,
             W0, b0, W1, b1, W2, b2):
    t0, t1, t2 = W0.shape[0], W1.shape[0], W2.shape[0]
    f32 = jnp.float32
    embT_spec = pl.BlockSpec((EMB, BB), lambda i: (0, i))
    idx_spec = pl.BlockSpec((1, BB), lambda i: (0, i))
    full = lambda shape: pl.BlockSpec(shape, lambda i: (0, 0))
    return pl.pallas_call(
        _tc_body,
        grid=(BATCH // BB,),
        in_specs=[idx_spec, idx_spec, idx_spec,
                  embT_spec, embT_spec, embT_spec,
                  full((EMB, t0)), full((t0, 1)),
                  full((EMB, t1)), full((1, t1)),
                  full((EMB, t2)), full((t2, 1))],
        out_specs=[pl.BlockSpec((t0, BB), lambda i: (0, i)),
                   pl.BlockSpec((BB, t1), lambda i: (i, 0)),
                   pl.BlockSpec((t2, BB), lambda i: (0, i))],
        out_shape=[jax.ShapeDtypeStruct((t0, BATCH), f32),
                   jax.ShapeDtypeStruct((BATCH, t1), f32),
                   jax.ShapeDtypeStruct((t2, BATCH), f32)],
    )(didx, c0idx, c1idx, dT_g, c0T_g, c1T_g,
      W0.T, b0.reshape(t0, 1), W1.T, b1.reshape(1, t1),
      W2.T, b2.reshape(t2, 1))


def kernel(dialects, chars, d_emb0, c_emb0, c_emb1, W0, b0, W1, b1, W2, b2):
    d_idx = dialects[:, 0].astype(jnp.int32)
    c0_idx = chars[:, 0].astype(jnp.int32)
    c1_idx = chars[:, 1].astype(jnp.int32)
    tails = jnp.concatenate([
        d_emb0[D_TF * 128:, :].T.reshape(-1),
        c_emb0[C_TF * 128:, :].T.reshape(-1),
        c_emb1[C_TF * 128:, :].T.reshape(-1)])
    g = _make_sc_gather()(
        d_idx >> 7, d_idx & 127, c0_idx >> 7, c0_idx & 127,
        c1_idx >> 7, c1_idx & 127, d_emb0.T, c_emb0.T, c_emb1.T, tails)
    d_g, c0_g, c1_g = (
        g[t].transpose(0, 2, 1, 3).reshape(EMB, BATCH) for t in range(3))
    o0T, o1, o2T = _tc_call(
        d_idx.reshape(1, BATCH), c0_idx.reshape(1, BATCH),
        c1_idx.reshape(1, BATCH), d_g, c0_g, c1_g,
        W0, b0, W1, b1, W2, b2)
    return (o0T.T, o1, o2T.T)
